# Initial kernel scaffold; baseline (speedup 1.0000x reference)
import jax
import jax.numpy as jnp
from jax.experimental import pallas as pl


def kernel(x, category_labels, labels, W_raise, gamma, beta, cls_W, cls_bias):
    raise NotImplementedError("write your pallas kernel here")


# speedup vs baseline: 13.3607x; 13.3607x over previous
import numpy as np
import jax
import jax.numpy as jnp
from jax.experimental import pallas as pl

_NUM_CAT = 16
_D = 128
_SEG = 6
_G = 8
_OUT_W = 64
_SEG_START = (0, 4, 6, 8, 12, 16, 19, 22, 24, 28, 30, 36, 38, 41, 44, 47)
_SEG_WIDTH = (4, 2, 2, 4, 4, 3, 3, 2, 4, 2, 6, 2, 3, 3, 3, 3)
_NEG = -1e30


def _make_constants():
    s = np.zeros((_NUM_CAT * _G, _G), np.float32)
    for c in range(_NUM_CAT):
        for k in range(_G):
            s[c * _G + k, k] = 1.0
    t = s.T.copy()
    p = np.zeros((_NUM_CAT * _G, _OUT_W), np.float32)
    for c in range(_NUM_CAT):
        for k in range(_SEG_WIDTH[c]):
            p[c * _G + k, _SEG_START[c] + k] = 1.0
    return s, t, p


_S_NP, _T_NP, _P_NP = _make_constants()


def _body(x_ref, cat_ref, wr_ref, gam_ref, bet_ref, wblk_ref, bias_ref,
          s_ref, t_ref, p_ref, out_ref):
    xb = x_ref[...]
    h = jnp.dot(xb, wr_ref[...], preferred_element_type=jnp.float32)
    h = gam_ref[...] * h + bet_ref[...]
    h = jnp.where(h >= 0.0, h, 0.2 * h)
    la = jnp.dot(h, wblk_ref[...], preferred_element_type=jnp.float32)
    cat = cat_ref[...]
    lane_cat = jax.lax.broadcasted_iota(jnp.int32, (1, _NUM_CAT * _G), 1) // _G
    mask = (cat == lane_cat)
    gated = jnp.where(mask, la, 0.0)
    logits = jnp.dot(gated, s_ref[...],
                     preferred_element_type=jnp.float32)
    logits = logits + bias_ref[...]
    m = jnp.max(logits, axis=1, keepdims=True)
    e = jnp.exp(logits - m)
    lse = m + jnp.log(jnp.sum(e, axis=1, keepdims=True))
    logsm = logits - lse
    tiled = jnp.dot(logsm, t_ref[...],
                    preferred_element_type=jnp.float32)
    g = jnp.where(mask, tiled, 0.0)
    out_ref[...] = jnp.dot(g, p_ref[...],
                           preferred_element_type=jnp.float32)


def kernel(x, category_labels, labels, W_raise, gamma, beta, cls_W, cls_bias):
    n = x.shape[0]
    tn = 1000
    grid = n // tn

    cat2 = category_labels.astype(jnp.int32).reshape(n, 1)
    gam2 = gamma.reshape(1, _NUM_CAT * _D)
    bet2 = beta.reshape(1, _NUM_CAT * _D)
    wblk = jnp.zeros((_NUM_CAT * _D, _NUM_CAT * _G), jnp.float32)
    for c in range(_NUM_CAT):
        wblk = wblk.at[c * _D:(c + 1) * _D, c * _G:c * _G + _SEG].set(cls_W[c])
    bias8 = jnp.concatenate(
        [cls_bias, jnp.full((_G - _SEG,), _NEG, jnp.float32)]).reshape(1, _G)

    out = pl.pallas_call(
        _body,
        grid=(grid,),
        in_specs=[
            pl.BlockSpec((tn, _D), lambda i: (i, 0)),
            pl.BlockSpec((tn, 1), lambda i: (i, 0)),
            pl.BlockSpec((_D, _NUM_CAT * _D), lambda i: (0, 0)),
            pl.BlockSpec((1, _NUM_CAT * _D), lambda i: (0, 0)),
            pl.BlockSpec((1, _NUM_CAT * _D), lambda i: (0, 0)),
            pl.BlockSpec((_NUM_CAT * _D, _NUM_CAT * _G), lambda i: (0, 0)),
            pl.BlockSpec((1, _G), lambda i: (0, 0)),
            pl.BlockSpec((_NUM_CAT * _G, _G), lambda i: (0, 0)),
            pl.BlockSpec((_G, _NUM_CAT * _G), lambda i: (0, 0)),
            pl.BlockSpec((_NUM_CAT * _G, _OUT_W), lambda i: (0, 0)),
        ],
        out_specs=pl.BlockSpec((tn, _OUT_W), lambda i: (i, 0)),
        out_shape=jax.ShapeDtypeStruct((n, _OUT_W), jnp.float32),
    )(x, cat2, W_raise, gam2, bet2, wblk, bias8,
      jnp.asarray(_S_NP), jnp.asarray(_T_NP), jnp.asarray(_P_NP))
    return out[:, :50]
